# baseline (speedup 1.0000x reference)
import functools
import math

import jax
import jax.numpy as jnp
import numpy as np
from jax import lax
from jax.experimental import pallas as pl
from jax.experimental.pallas import tpu as pltpu
from jax.experimental.pallas import tpu_sc as plsc

_NC = 2
_NS = 16
_NW = _NC * _NS
_LANES = 16
_NBUF = 4


@functools.lru_cache(maxsize=None)
def _positional_encoding(max_len, d_model):
    depth = d_model // 2
    positions = np.arange(max_len, dtype=np.float32)[:, None]
    depths = (np.arange(depth, dtype=np.float32) / float(depth))[None, :]
    angle_rates = (1.0 / (10000.0 ** depths)).astype(np.float32)
    angle_rads = positions * angle_rates
    return np.concatenate(
        [np.sin(angle_rads), np.cos(angle_rads)], axis=-1
    ).astype(np.float32)


@functools.partial(jax.jit, static_argnums=(3, 4, 5, 6))
def _run(sequences, table, pos, n_chunks, tchunk, bsz, d_model):
    chunk = bsz * tchunk
    t_len = pos.shape[0]
    t_per_w = n_chunks * tchunk
    total_rows = bsz * t_len
    scale = math.sqrt(float(d_model))
    n_vec = d_model // _LANES

    mesh = plsc.VectorSubcoreMesh(
        core_axis_name="c", subcore_axis_name="s", num_cores=_NC,
        num_subcores=_NS)

    scratch = (
        [pltpu.VMEM((bsz, t_per_w), jnp.int32)]
        + [pltpu.VMEM((chunk, d_model), jnp.float32)] * _NBUF
        + [pltpu.VMEM((tchunk, d_model), jnp.float32)] * _NBUF
        + [pltpu.SemaphoreType.DMA] * (3 * _NBUF)
    )

    @functools.partial(
        pl.kernel,
        out_type=jax.ShapeDtypeStruct((bsz, t_len, d_model), jnp.float32),
        mesh=mesh,
        scratch_types=scratch,
    )
    def k(seq_hbm, table_hbm, pos_hbm, out_hbm, idx_v, *rest):
        rows = list(rest[:_NBUF])
        pbuf = list(rest[_NBUF:2 * _NBUF])
        sem_p = list(rest[2 * _NBUF:3 * _NBUF])
        sem_g = list(rest[3 * _NBUF:4 * _NBUF])
        sem_o = list(rest[4 * _NBUF:5 * _NBUF])

        wid = lax.axis_index("s") * _NC + lax.axis_index("c")
        tbase = wid * t_per_w

        def pos_src(gg):
            return pos_hbm.at[pl.ds(tbase + gg * tchunk, tchunk)]

        def pos_start(gg, b):
            pltpu.async_copy(pos_src(gg), pbuf[b], sem_p[b])

        def pos_wait(gg, b):
            pltpu.make_async_copy(pos_src(gg), pbuf[b], sem_p[b]).wait()

        def gather_start(gg, b):
            for bb in range(bsz):
                pltpu.async_copy(
                    table_hbm.at[idx_v.at[bb, pl.ds(gg * tchunk, tchunk)]],
                    rows[b].at[pl.ds(bb * tchunk, tchunk)],
                    sem_g[b])

        def gather_wait(gg, b):
            for bb in range(bsz):
                pltpu.make_async_copy(
                    table_hbm.at[idx_v.at[bb, pl.ds(gg * tchunk, tchunk)]],
                    rows[b].at[pl.ds(bb * tchunk, tchunk)],
                    sem_g[b]).wait()

        def scatter_start(gg, b):
            for bb in range(bsz):
                pltpu.async_copy(
                    rows[b].at[pl.ds(bb * tchunk, tchunk)],
                    out_hbm.at[bb, pl.ds(tbase + gg * tchunk, tchunk)],
                    sem_o[b])

        def scatter_wait(gg, b):
            for bb in range(bsz):
                pltpu.make_async_copy(
                    rows[b].at[pl.ds(bb * tchunk, tchunk)],
                    out_hbm.at[bb, pl.ds(tbase + gg * tchunk, tchunk)],
                    sem_o[b]).wait()

        pltpu.sync_copy(seq_hbm.at[:, pl.ds(tbase, t_per_w)], idx_v)
        for b in range(_NBUF):
            pos_start(b, b)
        for b in range(_NBUF - 1):
            gather_start(b, b)

        @pl.loop(0, n_chunks, step=_NBUF)
        def _ring(g):
            for b in range(_NBUF):
                gg = g + b
                b3 = (b + 3) % _NBUF

                gather_wait(gg, b)
                pos_wait(gg, b)

                @pl.loop(0, tchunk)
                def _row(j):
                    for v in range(n_vec):
                        sl = pl.ds(v * _LANES, _LANES)
                        pv = pbuf[b][j, sl]
                        for bb in range(bsz):
                            r = bb * tchunk + j
                            rows[b][r, sl] = rows[b][r, sl] * scale + pv

                scatter_start(gg, b)

                @pl.when(gg + _NBUF < n_chunks)
                def _():
                    pos_start(gg + _NBUF, b)

                @pl.when(gg + _NBUF - 1 < n_chunks)
                def _():
                    @pl.when(gg >= 1)
                    def _():
                        scatter_wait(gg - 1, b3)

                    gather_start(gg + _NBUF - 1, b3)

        for b in range(_NBUF):
            scatter_wait(n_chunks - _NBUF + b, b)

    return k(sequences, table, pos)


def kernel(sequences, table):
    bsz, seq_len = sequences.shape
    vocab, d_model = table.shape
    tchunk = 8
    t_per_w = seq_len // _NW
    n_chunks = t_per_w // tchunk

    pos = _positional_encoding(seq_len, d_model)
    return _run(sequences.astype(jnp.int32), table, pos, n_chunks, tchunk,
                bsz, d_model)


# speedup vs baseline: 1.0095x; 1.0095x over previous
import functools
import math

import jax
import jax.numpy as jnp
import numpy as np
from jax import lax
from jax.experimental import pallas as pl
from jax.experimental.pallas import tpu as pltpu
from jax.experimental.pallas import tpu_sc as plsc

_NC = 2
_NS = 16
_NW = _NC * _NS
_LANES = 16
_NBUF = 4


@functools.lru_cache(maxsize=None)
def _positional_encoding(max_len, d_model):
    depth = d_model // 2
    positions = np.arange(max_len, dtype=np.float32)[:, None]
    depths = (np.arange(depth, dtype=np.float32) / float(depth))[None, :]
    angle_rates = (1.0 / (10000.0 ** depths)).astype(np.float32)
    angle_rads = positions * angle_rates
    return np.concatenate(
        [np.sin(angle_rads), np.cos(angle_rads)], axis=-1
    ).astype(np.float32)


@functools.partial(jax.jit, static_argnums=(3, 4, 5, 6))
def _run(sequences, table, pos_packed, n_chunks, tchunk, bsz, d_model):
    t_len = pos_packed.shape[0]
    t_per_w = n_chunks * tchunk
    scale = math.sqrt(float(d_model))
    n_vec = d_model // _LANES

    mesh = plsc.VectorSubcoreMesh(
        core_axis_name="c", subcore_axis_name="s", num_cores=_NC,
        num_subcores=_NS)

    scratch = (
        [pltpu.VMEM((bsz, t_per_w), jnp.int32)]
        + [pltpu.VMEM((bsz, tchunk, d_model), jnp.float32)] * _NBUF
        + [pltpu.VMEM((tchunk, d_model), jnp.float32)] * _NBUF
        + [pltpu.SemaphoreType.DMA] * (3 * _NBUF)
    )

    @functools.partial(
        pl.kernel,
        out_type=jax.ShapeDtypeStruct((bsz, t_len, d_model), jnp.float32),
        mesh=mesh,
        scratch_types=scratch,
    )
    def k(seq_hbm, table_hbm, pos_hbm, out_hbm, idx_v, *rest):
        rows = list(rest[:_NBUF])
        pbuf = list(rest[_NBUF:2 * _NBUF])
        sem_p = list(rest[2 * _NBUF:3 * _NBUF])
        sem_g = list(rest[3 * _NBUF:4 * _NBUF])
        sem_o = list(rest[4 * _NBUF:5 * _NBUF])

        wid = lax.axis_index("s") * _NC + lax.axis_index("c")
        tbase = wid * t_per_w

        def pos_src(gg):
            return pos_hbm.at[pl.ds(tbase + gg * tchunk, tchunk)]

        def pos_start(gg, b):
            pltpu.async_copy(pos_src(gg), pbuf[b], sem_p[b])

        def pos_wait(gg, b):
            pltpu.make_async_copy(pos_src(gg), pbuf[b], sem_p[b]).wait()

        def gather_start(gg, b):
            for bb in range(bsz):
                pltpu.async_copy(
                    table_hbm.at[idx_v.at[bb, pl.ds(gg * tchunk, tchunk)]],
                    rows[b].at[bb],
                    sem_g[b])

        def gather_wait(gg, b):
            for bb in range(bsz):
                pltpu.make_async_copy(
                    table_hbm.at[idx_v.at[bb, pl.ds(gg * tchunk, tchunk)]],
                    rows[b].at[bb],
                    sem_g[b]).wait()

        def out_dst(gg):
            return out_hbm.at[:, pl.ds(tbase + gg * tchunk, tchunk)]

        def scatter_start(gg, b):
            pltpu.async_copy(rows[b], out_dst(gg), sem_o[b])

        def scatter_wait(gg, b):
            pltpu.make_async_copy(rows[b], out_dst(gg), sem_o[b]).wait()

        pltpu.sync_copy(seq_hbm.at[:, pl.ds(tbase, t_per_w)], idx_v)
        for b in range(_NBUF):
            pos_start(b, b)
        for b in range(_NBUF - 1):
            gather_start(b, b)

        @pl.loop(0, n_chunks, step=_NBUF)
        def _ring(g):
            for b in range(_NBUF):
                gg = g + b
                b3 = (b + 3) % _NBUF

                gather_wait(gg, b)
                pos_wait(gg, b)

                @pl.loop(0, tchunk)
                def _row(j):
                    for v in range(n_vec):
                        sl = pl.ds(v * _LANES, _LANES)
                        pv = pbuf[b][j, sl]
                        for bb in range(bsz):
                            rows[b][bb, j, sl] = (
                                rows[b][bb, j, sl] * scale + pv)

                scatter_start(gg, b)

                @pl.when(gg + _NBUF < n_chunks)
                def _():
                    pos_start(gg + _NBUF, b)

                @pl.when(gg + _NBUF - 1 < n_chunks)
                def _():
                    @pl.when(gg >= 1)
                    def _():
                        scatter_wait(gg - 1, b3)

                    gather_start(gg + _NBUF - 1, b3)

        for b in range(_NBUF):
            scatter_wait(n_chunks - _NBUF + b, b)

    return k(sequences, table, pos_packed)


def kernel(sequences, table):
    bsz, seq_len = sequences.shape
    vocab, d_model = table.shape
    tchunk = 8
    t_per_w = seq_len // _NW
    n_chunks = t_per_w // tchunk

    pos = _positional_encoding(seq_len, d_model)
    return _run(sequences.astype(jnp.int32), table, pos, n_chunks,
                tchunk, bsz, d_model)


# speedup vs baseline: 1.0185x; 1.0090x over previous
import functools
import math

import jax
import jax.numpy as jnp
import numpy as np
from jax import lax
from jax.experimental import pallas as pl
from jax.experimental.pallas import tpu as pltpu
from jax.experimental.pallas import tpu_sc as plsc

_NC = 2
_NS = 16
_NW = _NC * _NS
_LANES = 16
_NBUF = 4


@functools.lru_cache(maxsize=None)
def _positional_encoding(max_len, d_model):
    depth = d_model // 2
    positions = np.arange(max_len, dtype=np.float32)[:, None]
    depths = (np.arange(depth, dtype=np.float32) / float(depth))[None, :]
    angle_rates = (1.0 / (10000.0 ** depths)).astype(np.float32)
    angle_rads = positions * angle_rates
    return np.concatenate(
        [np.sin(angle_rads), np.cos(angle_rads)], axis=-1
    ).astype(np.float32)


@functools.partial(jax.jit, static_argnums=(3, 4, 5, 6))
def _run(sequences, table, pos_packed, n_chunks, tchunk, bsz, d_model):
    t_len = pos_packed.shape[0]
    t_per_w = n_chunks * tchunk
    scale = math.sqrt(float(d_model))
    n_vec = d_model // _LANES

    mesh = plsc.VectorSubcoreMesh(
        core_axis_name="c", subcore_axis_name="s", num_cores=_NC,
        num_subcores=_NS)

    scratch = (
        [pltpu.VMEM((bsz, t_per_w), jnp.int32)]
        + [pltpu.VMEM((bsz, tchunk, d_model), jnp.float32)] * _NBUF
        + [pltpu.VMEM((tchunk, d_model), jnp.float32)] * _NBUF
        + [pltpu.SemaphoreType.DMA] * (3 * _NBUF)
    )

    @functools.partial(
        pl.kernel,
        out_type=jax.ShapeDtypeStruct((bsz, t_len, d_model), jnp.float32),
        mesh=mesh,
        scratch_types=scratch,
    )
    def k(seq_hbm, table_hbm, pos_hbm, out_hbm, idx_v, *rest):
        rows = list(rest[:_NBUF])
        pbuf = list(rest[_NBUF:2 * _NBUF])
        sem_p = list(rest[2 * _NBUF:3 * _NBUF])
        sem_g = list(rest[3 * _NBUF:4 * _NBUF])
        sem_o = list(rest[4 * _NBUF:5 * _NBUF])

        wid = lax.axis_index("s") * _NC + lax.axis_index("c")
        tbase = wid * t_per_w

        def pos_src(gg):
            return pos_hbm.at[pl.ds(tbase + gg * tchunk, tchunk)]

        def pos_start(gg, b):
            pltpu.async_copy(pos_src(gg), pbuf[b], sem_p[b])

        def pos_wait(gg, b):
            pltpu.make_async_copy(pos_src(gg), pbuf[b], sem_p[b]).wait()

        def gather_start(gg, b):
            for bb in range(bsz):
                pltpu.async_copy(
                    table_hbm.at[idx_v.at[bb, pl.ds(gg * tchunk, tchunk)]],
                    rows[b].at[bb],
                    sem_g[b])

        def gather_wait(gg, b):
            for bb in range(bsz):
                pltpu.make_async_copy(
                    table_hbm.at[idx_v.at[bb, pl.ds(gg * tchunk, tchunk)]],
                    rows[b].at[bb],
                    sem_g[b]).wait()

        def out_dst(gg):
            return out_hbm.at[:, pl.ds(tbase + gg * tchunk, tchunk)]

        def scatter_start(gg, b):
            pltpu.async_copy(rows[b], out_dst(gg), sem_o[b])

        def scatter_wait(gg, b):
            pltpu.make_async_copy(rows[b], out_dst(gg), sem_o[b]).wait()

        pltpu.sync_copy(seq_hbm.at[:, pl.ds(tbase, t_per_w)], idx_v)
        for b in range(_NBUF):
            pos_start(b, b)
        for b in range(_NBUF - 1):
            gather_start(b, b)

        @pl.loop(0, n_chunks, step=_NBUF)
        def _ring(g):
            for b in range(_NBUF):
                gg = g + b
                b3 = (b + 3) % _NBUF

                gather_wait(gg, b)
                pos_wait(gg, b)

                @pl.when(gg + _NBUF - 1 < n_chunks)
                def _():
                    @pl.when(gg >= 1)
                    def _():
                        scatter_wait(gg - 1, b3)

                    gather_start(gg + _NBUF - 1, b3)

                @pl.loop(0, tchunk)
                def _row(j):
                    for v in range(n_vec):
                        sl = pl.ds(v * _LANES, _LANES)
                        pv = pbuf[b][j, sl]
                        for bb in range(bsz):
                            rows[b][bb, j, sl] = (
                                rows[b][bb, j, sl] * scale + pv)

                scatter_start(gg, b)

                @pl.when(gg + _NBUF < n_chunks)
                def _():
                    pos_start(gg + _NBUF, b)

        for b in range(_NBUF):
            scatter_wait(n_chunks - _NBUF + b, b)

    return k(sequences, table, pos_packed)


def kernel(sequences, table):
    bsz, seq_len = sequences.shape
    vocab, d_model = table.shape
    tchunk = 8
    t_per_w = seq_len // _NW
    n_chunks = t_per_w // tchunk

    pos = _positional_encoding(seq_len, d_model)
    return _run(sequences.astype(jnp.int32), table, pos, n_chunks,
                tchunk, bsz, d_model)
